# baseline (speedup 1.0000x reference)
import functools

import jax
import jax.numpy as jnp
from jax import lax
from jax.experimental import pallas as pl
from jax.experimental.pallas import tpu as pltpu
from jax.experimental.pallas import tpu_sc as plsc

N_FEATURES = 40960
BATCH = 16384
L = 32

NUM_CORES = 2
NUM_SUBCORES = 16
NUM_WORKERS = NUM_CORES * NUM_SUBCORES
ROWS_PER_W = BATCH // NUM_WORKERS
TBL = N_FEATURES + 1


def _psqt_body(ics_hbm, tbl_hbm, out_hbm, idx_v, tbl_v, out_v, sem_i, sem_t):
    wid = lax.axis_index("s") * NUM_CORES + lax.axis_index("c")
    row_base = wid * ROWS_PER_W

    cp_t = pltpu.async_copy(tbl_hbm, tbl_v, sem_t)
    cp_i = pltpu.async_copy(ics_hbm.at[:, pl.ds(row_base, ROWS_PER_W)], idx_v,
                            sem_i)
    cp_i.wait()
    cp_t.wait()

    @plsc.parallel_loop(0, ROWS_PER_W // 16, unroll=2)
    def group(g):
        base = g * 16
        acc = [jnp.zeros((16,), jnp.float32) for _ in range(4)]
        for l in range(L):
            idx = idx_v[l, pl.ds(base, 16)] + 1
            acc[l % 4] = acc[l % 4] + plsc.load_gather(tbl_v, [idx])
        out_v[pl.ds(base, 16)] = (acc[0] + acc[1]) + (acc[2] + acc[3])
    pltpu.sync_copy(out_v, out_hbm.at[pl.ds(row_base, ROWS_PER_W)])


@jax.jit
def kernel(ics, weight):
    ics_t = ics.T
    tbl = weight[:, 0]
    mesh = plsc.VectorSubcoreMesh(core_axis_name="c", subcore_axis_name="s")
    out = pl.kernel(
        _psqt_body,
        out_type=jax.ShapeDtypeStruct((BATCH,), jnp.float32),
        mesh=mesh,
        scratch_types=[
            pltpu.VMEM((L, ROWS_PER_W), jnp.int32),
            pltpu.VMEM((TBL,), jnp.float32),
            pltpu.VMEM((ROWS_PER_W,), jnp.float32),
            pltpu.SemaphoreType.DMA,
            pltpu.SemaphoreType.DMA,
        ],
        compiler_params=pltpu.CompilerParams(
            needs_layout_passes=False, use_tc_tiling_on_sc=True),
    )(ics_t, tbl)
    return out.reshape(BATCH, 1)


# speedup vs baseline: 1.0264x; 1.0264x over previous
import functools

import jax
import jax.numpy as jnp
from jax import lax
from jax.experimental import pallas as pl
from jax.experimental.pallas import tpu as pltpu
from jax.experimental.pallas import tpu_sc as plsc

N_FEATURES = 40960
BATCH = 16384
L = 32

NUM_CORES = 2
NUM_SUBCORES = 16
NUM_WORKERS = NUM_CORES * NUM_SUBCORES
ROWS_PER_W = BATCH // NUM_WORKERS
TBL = N_FEATURES + 1


def _psqt_body(ics_hbm, tbl_hbm, out_hbm, idx_v, tbl_v, out_v, sem_i, sem_t):
    wid = lax.axis_index("s") * NUM_CORES + lax.axis_index("c")
    row_base = wid * ROWS_PER_W

    cp_t = pltpu.async_copy(tbl_hbm, tbl_v, sem_t)
    cp_i = pltpu.async_copy(ics_hbm.at[:, pl.ds(row_base, ROWS_PER_W)], idx_v,
                            sem_i)
    cp_i.wait()
    cp_t.wait()

    def group(g, carry):
        base = g * 16
        acc = [jnp.zeros((16,), jnp.float32) for _ in range(4)]
        for l in range(L):
            idx = idx_v[l, pl.ds(base, 16)] + 1
            acc[l % 4] = acc[l % 4] + plsc.load_gather(tbl_v, [idx])
        out_v[pl.ds(base, 16)] = (acc[0] + acc[1]) + (acc[2] + acc[3])
        return carry

    lax.fori_loop(0, ROWS_PER_W // 16, group, 0, unroll=4)
    pltpu.sync_copy(out_v, out_hbm.at[pl.ds(row_base, ROWS_PER_W)])


@jax.jit
def kernel(ics, weight):
    ics_t = ics.T
    tbl = weight[:, 0]
    mesh = plsc.VectorSubcoreMesh(core_axis_name="c", subcore_axis_name="s")
    out = pl.kernel(
        _psqt_body,
        out_type=jax.ShapeDtypeStruct((BATCH,), jnp.float32),
        mesh=mesh,
        scratch_types=[
            pltpu.VMEM((L, ROWS_PER_W), jnp.int32),
            pltpu.VMEM((TBL,), jnp.float32),
            pltpu.VMEM((ROWS_PER_W,), jnp.float32),
            pltpu.SemaphoreType.DMA,
            pltpu.SemaphoreType.DMA,
        ],
        compiler_params=pltpu.CompilerParams(
            needs_layout_passes=False, use_tc_tiling_on_sc=True),
    )(ics_t, tbl)
    return out.reshape(BATCH, 1)


# speedup vs baseline: 1.0384x; 1.0117x over previous
import functools

import jax
import jax.numpy as jnp
from jax import lax
from jax.experimental import pallas as pl
from jax.experimental.pallas import tpu as pltpu
from jax.experimental.pallas import tpu_sc as plsc

N_FEATURES = 40960
BATCH = 16384
L = 32

NUM_CORES = 2
NUM_SUBCORES = 16
NUM_WORKERS = NUM_CORES * NUM_SUBCORES
ROWS_PER_W = BATCH // NUM_WORKERS
TBL = N_FEATURES + 1


def _psqt_body(ics_hbm, tbl_hbm, out_hbm, idx_v, tbl_v, out_v, sem_i, sem_t):
    wid = lax.axis_index("s") * NUM_CORES + lax.axis_index("c")
    row_base = wid * ROWS_PER_W

    cp_t = pltpu.async_copy(tbl_hbm, tbl_v, sem_t)
    cp_i = pltpu.async_copy(ics_hbm.at[:, pl.ds(row_base, ROWS_PER_W)], idx_v,
                            sem_i)
    cp_i.wait()
    cp_t.wait()

    def group(g, carry):
        base = g * 16
        acc = [jnp.zeros((16,), jnp.float32) for _ in range(4)]
        for l in range(L):
            idx = idx_v[l, pl.ds(base, 16)] + 1
            acc[l % 4] = acc[l % 4] + plsc.load_gather(tbl_v, [idx])
        out_v[pl.ds(base, 16)] = (acc[0] + acc[1]) + (acc[2] + acc[3])
        return carry

    lax.fori_loop(0, ROWS_PER_W // 16, group, 0)
    pltpu.sync_copy(out_v, out_hbm.at[pl.ds(row_base, ROWS_PER_W)])


@jax.jit
def kernel(ics, weight):
    ics_t = ics.T
    tbl = weight[:, 0]
    mesh = plsc.VectorSubcoreMesh(core_axis_name="c", subcore_axis_name="s")
    out = pl.kernel(
        _psqt_body,
        out_type=jax.ShapeDtypeStruct((BATCH,), jnp.float32),
        mesh=mesh,
        scratch_types=[
            pltpu.VMEM((L, ROWS_PER_W), jnp.int32),
            pltpu.VMEM((TBL,), jnp.float32),
            pltpu.VMEM((ROWS_PER_W,), jnp.float32),
            pltpu.SemaphoreType.DMA,
            pltpu.SemaphoreType.DMA,
        ],
        compiler_params=pltpu.CompilerParams(
            needs_layout_passes=False, use_tc_tiling_on_sc=True),
    )(ics_t, tbl)
    return out.reshape(BATCH, 1)
